# Initial kernel scaffold; baseline (speedup 1.0000x reference)
import jax
import jax.numpy as jnp
from jax.experimental import pallas as pl


def kernel(x, edge_index, W1, b1, gamma, beta, bn_mean, bn_var, Wmix, bmix):
    raise NotImplementedError("write your pallas kernel here")


# speedup vs baseline: 17.6985x; 17.6985x over previous
import functools

import jax
import jax.numpy as jnp
from jax import lax
from jax.experimental import pallas as pl
from jax.experimental.pallas import tpu as pltpu
from jax.experimental.pallas import tpu_sc as plsc

RW = 8
NG = 3
SUBW = 128
SUB = 8
SUPER = SUB * SUBW


def _sc_mesh():
    return plsc.VectorSubcoreMesh(core_axis_name="c", subcore_axis_name="s")


def _num_cores_subcores():
    try:
        info = plsc.get_sparse_core_info()
        return info.num_cores, info.num_subcores
    except Exception:
        return 2, 16


def _make_prop(n, nacc, nsup, nc, ns):
    zblks = nacc // (ns * SUBW)
    wb = nacc // ns
    nsup2 = nsup // nc

    @functools.partial(
        pl.kernel,
        out_type=jax.ShapeDtypeStruct((nc, NG, nacc, RW), jnp.float32),
        mesh=_sc_mesh(),
        compiler_params=pltpu.CompilerParams(use_tc_tiling_on_sc=False),
        scratch_types=[
            pltpu.VMEM((SUB, SUBW), jnp.int32),
            pltpu.VMEM((SUB, SUBW), jnp.int32),
            pltpu.VMEM((SUB, SUBW, RW), jnp.float32),
            pltpu.VMEM((SUBW, RW), jnp.float32),
            pltpu.VMEM_SHARED((nacc, RW), jnp.float32),
            pltpu.SemaphoreType.DMA,
        ],
    )
    def prop(g_hbm, srcr_hbm, dstr_hbm, zero_hbm, out_hbm,
             src_v, dst_v, rows_v, zero_v, acc_sh, gsem):
        c = lax.axis_index("c")
        s = lax.axis_index("s")
        pltpu.sync_copy(zero_hbm, zero_v)

        for g in range(NG):
            def zbody(r, carry):
                pltpu.sync_copy(
                    zero_v, acc_sh.at[pl.ds((s * zblks + r) * SUBW, SUBW)])
                return carry

            lax.fori_loop(0, zblks, zbody, 0)
            plsc.subcore_barrier()

            def body(i, carry):
                row0 = (s * nsup + c * nsup2 + i) * SUB
                pltpu.sync_copy(srcr_hbm.at[g, pl.ds(row0, SUB)], src_v)
                pltpu.sync_copy(dstr_hbm.at[pl.ds(row0, SUB)], dst_v)
                descs = []
                for j in range(SUB):
                    descs.append(
                        pltpu.async_copy(g_hbm.at[src_v.at[j]],
                                         rows_v.at[j], gsem))
                for j in range(SUB):
                    descs[j].wait()
                for j in range(SUB):
                    pltpu.sync_copy(rows_v.at[j], acc_sh.at[dst_v.at[j]],
                                    add=True)
                return carry

            lax.fori_loop(0, nsup2, body, 0)
            plsc.subcore_barrier()
            pltpu.sync_copy(acc_sh.at[pl.ds(s * wb, wb)],
                            out_hbm.at[c, g, pl.ds(s * wb, wb)])
            plsc.subcore_barrier()

    return prop


def _make_deg(n, nacc, nsupd, nc, ns):
    zblks = nacc // (ns * SUBW)
    wb = nacc // ns

    @functools.partial(
        pl.kernel,
        out_type=jax.ShapeDtypeStruct((nc, nacc, RW), jnp.float32),
        mesh=_sc_mesh(),
        compiler_params=pltpu.CompilerParams(use_tc_tiling_on_sc=False),
        scratch_types=[
            pltpu.VMEM((SUB, SUBW), jnp.int32),
            pltpu.VMEM((SUBW, RW), jnp.float32),
            pltpu.VMEM((SUBW, RW), jnp.float32),
            pltpu.VMEM_SHARED((nacc, RW), jnp.float32),
        ],
    )
    def deg(srcr_hbm, ones_hbm, zero_hbm, out_hbm,
            idx_v, ones_v, zero_v, acc_sh):
        c = lax.axis_index("c")
        s = lax.axis_index("s")

        pltpu.sync_copy(zero_hbm, zero_v)
        pltpu.sync_copy(ones_hbm, ones_v)

        def zbody(r, carry):
            pltpu.sync_copy(zero_v,
                            acc_sh.at[pl.ds((s * zblks + r) * SUBW, SUBW)])
            return carry

        lax.fori_loop(0, zblks, zbody, 0)
        plsc.subcore_barrier()

        def body(i, carry):
            row0 = ((c * ns + s) * nsupd + i) * SUB
            pltpu.sync_copy(srcr_hbm.at[pl.ds(row0, SUB)], idx_v)
            for j in range(SUB):
                pltpu.sync_copy(ones_v, acc_sh.at[idx_v.at[j]], add=True)
            return carry

        lax.fori_loop(0, nsupd, body, 0)
        plsc.subcore_barrier()
        pltpu.sync_copy(acc_sh.at[pl.ds(s * wb, wb)],
                        out_hbm.at[c, pl.ds(s * wb, wb)])

    return deg


def _prep0_body(x_ref, dp_ref, w0_ref, dinv_ref, g_ref, t0_ref, out_ref):
    deg = dp_ref[0, :, 0:1] + dp_ref[1, :, 0:1]
    dinv = jnp.where(deg > 0.0,
                     lax.rsqrt(jnp.maximum(deg, 1e-12)),
                     jnp.zeros_like(deg))
    dinv_ref[...] = dinv
    xb = x_ref[...]
    for g in range(NG):
        hg = xb[:, g * RW:(g + 1) * RW]
        t0_ref[g] = hg
        g_ref[g] = dinv * hg
    out_ref[...] = jnp.dot(xb, w0_ref[...],
                           preferred_element_type=jnp.float32)


def _step_body(first, with_g, *refs):
    if first:
        acc_ref, dinv_ref, wp_ref, outin_ref = refs[:4]
        orefs = refs[4:]
        tp2_ref = None
        scale = 1.0
    else:
        acc_ref, dinv_ref, tp2_ref, wp_ref, outin_ref = refs[:5]
        orefs = refs[5:]
        scale = 2.0
    if with_g:
        tk_ref, g_ref, outo_ref = orefs
    else:
        tk_ref, outo_ref = orefs
    dinv = dinv_ref[...]
    wp = wp_ref[...]
    o = outin_ref[...]
    for g in range(NG):
        acc_g = acc_ref[0, g] + acc_ref[1, g]
        t_g = (-scale) * dinv * acc_g
        if not first:
            t_g = t_g - tp2_ref[g]
        tk_ref[g] = t_g
        if with_g:
            g_ref[g] = dinv * t_g
        o = o + jnp.dot(t_g, wp[g * RW:(g + 1) * RW],
                        preferred_element_type=jnp.float32)
    outo_ref[...] = o


def _final_body(h_ref, b1_ref, gam_ref, bet_ref, mu_ref, var_ref,
                wm_ref, bm_ref, y_ref):
    o = h_ref[...] + b1_ref[...]
    o = (o - mu_ref[...]) * lax.rsqrt(var_ref[...] + 1e-5) * gam_ref[...] \
        + bet_ref[...]
    h = jnp.maximum(o, 0.0)
    y_ref[...] = jnp.dot(h, wm_ref[...],
                         preferred_element_type=jnp.float32) + bm_ref[...]


def kernel(x, edge_index, W1, b1, gamma, beta, bn_mean, bn_var, Wmix, bmix):
    n, n_in = x.shape
    e = edge_index.shape[1]
    kblk = W1.shape[0]
    n_hid = W1.shape[2]
    n_out = Wmix.shape[2]
    nc, ns = _num_cores_subcores()

    per_tile = ns * SUPER
    nsup = -(-e // per_tile)
    nsup = -(-nsup // nc) * nc
    nsupd = nsup // nc
    epad = ns * nsup * SUPER
    pad = epad - e

    src = edge_index[0]
    dst = edge_index[1]
    src_g = jnp.concatenate([src, jnp.zeros((pad,), jnp.int32)])
    src_n = jnp.concatenate([src, jnp.full((pad,), n, jnp.int32)])
    dst_n = jnp.concatenate([dst, jnp.full((pad,), n, jnp.int32)])
    src3r = jnp.stack([src_g + g * n for g in range(NG)]) \
               .reshape(NG, epad // SUBW, SUBW)
    dstr = dst_n.reshape(epad // SUBW, SUBW)
    srcdr = src_n.reshape(epad // SUBW, SUBW)

    nacc = -(-(n + 1) // (ns * SUBW)) * (ns * SUBW)

    zero128 = jnp.zeros((SUBW, RW), jnp.float32)
    ones128 = jnp.ones((SUBW, RW), jnp.float32)

    deg_fn = _make_deg(n, nacc, nsupd, nc, ns)
    prop_fn = _make_prop(n, nacc, nsup, nc, ns)

    degp = deg_fn(srcdr, ones128, zero128)

    bsz = 2000
    grid = (n // bsz,)
    f32 = jnp.float32

    spec_pack = pl.BlockSpec((NG, bsz, RW), lambda b: (0, b, 0))
    spec_acc = pl.BlockSpec((nc, NG, bsz, RW), lambda b: (0, 0, b, 0))
    spec_deg = pl.BlockSpec((nc, bsz, RW), lambda b: (0, b, 0))
    spec_x = pl.BlockSpec((bsz, n_in), lambda b: (b, 0))
    spec_dinv = pl.BlockSpec((bsz, 1), lambda b: (b, 0))
    spec_out = pl.BlockSpec((bsz, n_hid), lambda b: (b, 0))
    spec_w = pl.BlockSpec((n_in, n_hid), lambda b: (0, 0))

    dinv, g, t_prev2, out = pl.pallas_call(
        _prep0_body,
        grid=grid,
        in_specs=[spec_x, spec_deg, spec_w],
        out_specs=[spec_dinv, spec_pack, spec_pack, spec_out],
        out_shape=[
            jax.ShapeDtypeStruct((n, 1), f32),
            jax.ShapeDtypeStruct((NG, n, RW), f32),
            jax.ShapeDtypeStruct((NG, n, RW), f32),
            jax.ShapeDtypeStruct((n, n_hid), f32),
        ],
    )(x, degp, W1[0])

    t_prev1 = None
    for k in range(1, kblk):
        acc = prop_fn(g.reshape(NG * n, RW), src3r, dstr, zero128)
        first = (k == 1)
        with_g = (k < kblk - 1)
        out_shapes = [jax.ShapeDtypeStruct((NG, n, RW), f32)]
        out_specs = [spec_pack]
        if with_g:
            out_shapes.append(jax.ShapeDtypeStruct((NG, n, RW), f32))
            out_specs.append(spec_pack)
        out_shapes.append(jax.ShapeDtypeStruct((n, n_hid), f32))
        out_specs.append(spec_out)
        if first:
            in_specs = [spec_acc, spec_dinv, spec_w, spec_out]
            operands = (acc, dinv, W1[k], out)
            alias = {3: len(out_shapes) - 1}
        else:
            in_specs = [spec_acc, spec_dinv, spec_pack, spec_w, spec_out]
            operands = (acc, dinv, t_prev2, W1[k], out)
            alias = {4: len(out_shapes) - 1}
        res = pl.pallas_call(
            functools.partial(_step_body, first, with_g),
            grid=grid,
            in_specs=in_specs,
            out_specs=out_specs,
            out_shape=out_shapes,
            input_output_aliases=alias,
        )(*operands)
        if with_g:
            t_k, g, out = res
        else:
            t_k, out = res
        if first:
            t_prev1 = t_k
        else:
            t_prev2, t_prev1 = t_prev1, t_k

    spec_vec = pl.BlockSpec((1, n_hid), lambda b: (0, 0))
    spec_wm = pl.BlockSpec((n_hid, n_out), lambda b: (0, 0))
    spec_bm = pl.BlockSpec((1, n_out), lambda b: (0, 0))
    spec_y = pl.BlockSpec((bsz, n_out), lambda b: (b, 0))

    y = pl.pallas_call(
        _final_body,
        grid=grid,
        in_specs=[spec_out, spec_vec, spec_vec, spec_vec, spec_vec,
                  spec_vec, spec_wm, spec_bm],
        out_specs=spec_y,
        out_shape=jax.ShapeDtypeStruct((n, n_out), f32),
    )(out, b1.reshape(1, n_hid), gamma.reshape(1, n_hid),
      beta.reshape(1, n_hid), bn_mean.reshape(1, n_hid),
      bn_var.reshape(1, n_hid), Wmix[0], bmix.reshape(1, n_out))
    return y
